# baseline (speedup 1.0000x reference)
import functools

import jax
import jax.numpy as jnp
from jax.experimental import pallas as pl
from jax.experimental.pallas import tpu as pltpu


def _round_up(x, m):
    return ((x + m - 1) // m) * m


def _fused_kernel(
    ids_sref, table_ref, rel_ref, g_ref, b_ref,
    out_ref, rel_out_ref,
    raw_ref, tbl_ref, tile_a_ref, tile_b_ref, sem,
    *, tq, p, v, stride, n_tiles, eps,
):
    t = pl.program_id(0)
    slot = jax.lax.rem(t, 2)

    @pl.when(t == 0)
    def _prologue():
        cp = pltpu.make_async_copy(table_ref, raw_ref, sem)
        cp.start()

        xr = rel_ref[...]
        mean = jnp.mean(xr, axis=-1, keepdims=True)
        centered = xr - mean
        var = jnp.mean(centered * centered, axis=-1, keepdims=True)
        rel_out_ref[...] = (
            centered * jax.lax.rsqrt(var + eps) * g_ref[...] + b_ref[...]
        )

        cp.wait()
        for j in range(p):
            tbl_ref[j : j + p * v : p, :] = raw_ref[:, 128 * j : 128 * (j + 1)]

    def _gather_into(dst_ref):
        base = t * tq
        for mi in range(tq):
            idx = pl.multiple_of(ids_sref[base + mi], p)
            slab = tbl_ref[pl.ds(idx, p), :]
            dst_ref[mi : mi + p * stride : stride, :] = slab

    def _ln_from(src_ref):
        h_inv = 1.0 / (p * 128)
        for g in range(tq // 8):
            xg = jnp.concatenate(
                [src_ref[pl.ds(j * stride + 8 * g, 8), :] for j in range(p)],
                axis=1,
            )
            s1 = jnp.sum(xg, axis=-1, keepdims=True)
            s2 = jnp.sum(xg * xg, axis=-1, keepdims=True)
            mean = s1 * h_inv
            var = s2 * h_inv - mean * mean
            inv = jax.lax.rsqrt(var + eps)
            out_ref[pl.ds(8 * g, 8), :] = xg * inv - mean * inv

    @pl.when(slot == 0)
    def _even():
        _gather_into(tile_a_ref)
        _ln_from(tile_b_ref)

    @pl.when(slot == 1)
    def _odd():
        _gather_into(tile_b_ref)
        _ln_from(tile_a_ref)


def _embeddings_fused(input_ids, emb_table, rel_emb, gamma, beta, *, eps, tq=512):
    B, S = input_ids.shape
    V, H = emb_table.shape
    R, _ = rel_emb.shape
    N = B * S
    assert H % 128 == 0, "hidden size must be lane-tile aligned"
    p = H // 128

    tq_eff = min(tq, _round_up(N, 8))
    n_pad = _round_up(N, tq_eff)
    stride = tq_eff + 8

    ids = jnp.clip(input_ids.reshape(N).astype(jnp.int32), 0, V - 1) * p
    ids = jnp.pad(ids, (0, n_pad - N + tq_eff))

    table_bytes = V * H * 4
    vmem_limit = min(
        2 * table_bytes + 12 * tq_eff * H * 4 + (12 << 20),
        60 << 20,
    )

    n_tiles = n_pad // tq_eff
    grid_spec = pltpu.PrefetchScalarGridSpec(
        num_scalar_prefetch=1,
        grid=(n_tiles + 1,),
        in_specs=[
            pl.BlockSpec(memory_space=pl.ANY),
            pl.BlockSpec((R, H), lambda i, ids_ref: (0, 0)),
            pl.BlockSpec((1, H), lambda i, ids_ref: (0, 0)),
            pl.BlockSpec((1, H), lambda i, ids_ref: (0, 0)),
        ],
        out_specs=[
            pl.BlockSpec(
                (tq_eff, H), lambda i, ids_ref: (jnp.maximum(i - 1, 0), 0)
            ),
            pl.BlockSpec((R, H), lambda i, ids_ref: (0, 0)),
        ],
        scratch_shapes=[
            pltpu.VMEM((V, H), jnp.float32),
            pltpu.VMEM((V * p, 128), jnp.float32),
            pltpu.VMEM((stride * p, 128), jnp.float32),
            pltpu.VMEM((stride * p, 128), jnp.float32),
            pltpu.SemaphoreType.DMA,
        ],
    )

    word, rel = pl.pallas_call(
        functools.partial(
            _fused_kernel,
            tq=tq_eff, p=p, v=V, stride=stride, n_tiles=n_tiles, eps=eps,
        ),
        out_shape=[
            jax.ShapeDtypeStruct((n_pad, H), jnp.float32),
            jax.ShapeDtypeStruct((R, H), jnp.float32),
        ],
        grid_spec=grid_spec,
        compiler_params=pltpu.CompilerParams(
            dimension_semantics=("arbitrary",),
            vmem_limit_bytes=vmem_limit,
        ),
    )(ids, emb_table, rel_emb, gamma.reshape(1, H), beta.reshape(1, H))
    return word[:N].reshape(B, S, H), rel


def kernel(input_ids, word_emb, rel_emb, rel_gamma, rel_beta):
    return _embeddings_fused(
        input_ids, word_emb, rel_emb, rel_gamma, rel_beta, eps=1e-7
    )


# speedup vs baseline: 1.1483x; 1.1483x over previous
import functools

import jax
import jax.numpy as jnp
from jax.experimental import pallas as pl
from jax.experimental.pallas import tpu as pltpu


def _round_up(x, m):
    return ((x + m - 1) // m) * m


def _fused_kernel(
    ids_sref, table_ref, rel_ref, g_ref, b_ref,
    out_ref, rel_out_ref,
    raw_ref, tbl_ref, tile_a_ref, tile_b_ref, sem,
    *, tq, p, v, stride, n_tiles, eps,
):
    t = pl.program_id(0)
    slot = jax.lax.rem(t, 2)

    @pl.when(t == 0)
    def _prologue():
        nq = 4
        qv = v // nq
        cps = [
            pltpu.make_async_copy(
                table_ref.at[pl.ds(q * qv, qv), :],
                raw_ref.at[pl.ds(q * qv, qv), :],
                sem.at[q],
            )
            for q in range(nq)
        ]
        for cp in cps:
            cp.start()

        xr = rel_ref[...]
        mean = jnp.mean(xr, axis=-1, keepdims=True)
        centered = xr - mean
        var = jnp.mean(centered * centered, axis=-1, keepdims=True)
        rel_out_ref[...] = (
            centered * jax.lax.rsqrt(var + eps) * g_ref[...] + b_ref[...]
        )

        for q in range(nq):
            cps[q].wait()
            for j in range(p):
                tbl_ref[j + q * qv * p : j + (q + 1) * qv * p : p, :] = (
                    raw_ref[pl.ds(q * qv, qv), 128 * j : 128 * (j + 1)]
                )

    def _gather_into(dst_ref):
        base = t * tq
        for mi in range(tq):
            idx = pl.multiple_of(ids_sref[base + mi], p)
            slab = tbl_ref[pl.ds(idx, p), :]
            dst_ref[mi : mi + p * stride : stride, :] = slab

    def _ln_from(src_ref):
        h_inv = 1.0 / (p * 128)
        for g in range(tq // 8):
            xg = jnp.concatenate(
                [src_ref[pl.ds(j * stride + 8 * g, 8), :] for j in range(p)],
                axis=1,
            )
            s1 = jnp.sum(xg, axis=-1, keepdims=True)
            s2 = jnp.sum(xg * xg, axis=-1, keepdims=True)
            mean = s1 * h_inv
            var = s2 * h_inv - mean * mean
            inv = jax.lax.rsqrt(var + eps)
            out_ref[pl.ds(8 * g, 8), :] = xg * inv - mean * inv

    @pl.when(slot == 0)
    def _even():
        _gather_into(tile_a_ref)
        _ln_from(tile_b_ref)

    @pl.when(slot == 1)
    def _odd():
        _gather_into(tile_b_ref)
        _ln_from(tile_a_ref)


def _embeddings_fused(input_ids, emb_table, rel_emb, gamma, beta, *, eps, tq=1024):
    B, S = input_ids.shape
    V, H = emb_table.shape
    R, _ = rel_emb.shape
    N = B * S
    assert H % 128 == 0, "hidden size must be lane-tile aligned"
    p = H // 128

    tq_eff = min(tq, _round_up(N, 8))
    n_pad = _round_up(N, tq_eff)
    stride = tq_eff + 8

    ids = jnp.clip(input_ids.reshape(N).astype(jnp.int32), 0, V - 1) * p
    ids = jnp.pad(ids, (0, n_pad - N + tq_eff))

    table_bytes = V * H * 4
    vmem_limit = min(
        2 * table_bytes + 12 * tq_eff * H * 4 + (12 << 20),
        60 << 20,
    )

    n_tiles = n_pad // tq_eff
    grid_spec = pltpu.PrefetchScalarGridSpec(
        num_scalar_prefetch=1,
        grid=(n_tiles + 1,),
        in_specs=[
            pl.BlockSpec(memory_space=pl.ANY),
            pl.BlockSpec((R, H), lambda i, ids_ref: (0, 0)),
            pl.BlockSpec((1, H), lambda i, ids_ref: (0, 0)),
            pl.BlockSpec((1, H), lambda i, ids_ref: (0, 0)),
        ],
        out_specs=[
            pl.BlockSpec(
                (tq_eff, H), lambda i, ids_ref: (jnp.maximum(i - 1, 0), 0)
            ),
            pl.BlockSpec((R, H), lambda i, ids_ref: (0, 0)),
        ],
        scratch_shapes=[
            pltpu.VMEM((V, H), jnp.float32),
            pltpu.VMEM((V * p, 128), jnp.float32),
            pltpu.VMEM((stride * p, 128), jnp.float32),
            pltpu.VMEM((stride * p, 128), jnp.float32),
            pltpu.SemaphoreType.DMA((4,)),
        ],
    )

    word, rel = pl.pallas_call(
        functools.partial(
            _fused_kernel,
            tq=tq_eff, p=p, v=V, stride=stride, n_tiles=n_tiles, eps=eps,
        ),
        out_shape=[
            jax.ShapeDtypeStruct((n_pad, H), jnp.float32),
            jax.ShapeDtypeStruct((R, H), jnp.float32),
        ],
        grid_spec=grid_spec,
        compiler_params=pltpu.CompilerParams(
            dimension_semantics=("arbitrary",),
            vmem_limit_bytes=vmem_limit,
        ),
    )(ids, emb_table, rel_emb, gamma.reshape(1, H), beta.reshape(1, H))
    return word[:N].reshape(B, S, H), rel


def kernel(input_ids, word_emb, rel_emb, rel_gamma, rel_beta):
    return _embeddings_fused(
        input_ids, word_emb, rel_emb, rel_gamma, rel_beta, eps=1e-7
    )


# speedup vs baseline: 1.1586x; 1.0090x over previous
import functools

import jax
import jax.numpy as jnp
from jax.experimental import pallas as pl
from jax.experimental.pallas import tpu as pltpu


def _round_up(x, m):
    return ((x + m - 1) // m) * m


def _fused_kernel(
    ids_sref, table_ref, rel_ref, g_ref, b_ref,
    out_ref, rel_out_ref,
    raw_ref, tbl_ref, tile_a_ref, tile_b_ref, sem,
    *, tq, p, v, stride, n_tiles, eps,
):
    t = pl.program_id(0)
    slot = jax.lax.rem(t, 2)

    @pl.when(t == 0)
    def _prologue():
        nq = 8
        qv = v // nq
        cps = [
            pltpu.make_async_copy(
                table_ref.at[pl.ds(q * qv, qv), :],
                raw_ref.at[pl.ds(q * qv, qv), :],
                sem.at[q],
            )
            for q in range(nq)
        ]
        for cp in cps:
            cp.start()

        xr = rel_ref[...]
        mean = jnp.mean(xr, axis=-1, keepdims=True)
        centered = xr - mean
        var = jnp.mean(centered * centered, axis=-1, keepdims=True)
        rel_out_ref[...] = (
            centered * jax.lax.rsqrt(var + eps) * g_ref[...] + b_ref[...]
        )

        for q in range(nq):
            cps[q].wait()
            for j in range(p):
                tbl_ref[j + q * qv * p : j + (q + 1) * qv * p : p, :] = (
                    raw_ref[pl.ds(q * qv, qv), 128 * j : 128 * (j + 1)]
                )

    def _gather_into(dst_ref):
        base = t * tq
        for mi in range(tq):
            idx = pl.multiple_of(ids_sref[base + mi], p)
            slab = tbl_ref[pl.ds(idx, p), :]
            dst_ref[mi : mi + p * stride : stride, :] = slab

    def _ln_from(src_ref):
        h_inv = 1.0 / (p * 128)
        for g in range(tq // 8):
            xg = jnp.concatenate(
                [src_ref[pl.ds(j * stride + 8 * g, 8), :] for j in range(p)],
                axis=1,
            )
            s1 = jnp.sum(xg, axis=-1, keepdims=True)
            s2 = jnp.sum(xg * xg, axis=-1, keepdims=True)
            mean = s1 * h_inv
            var = s2 * h_inv - mean * mean
            inv = jax.lax.rsqrt(var + eps)
            out_ref[pl.ds(8 * g, 8), :] = xg * inv - mean * inv

    @pl.when(slot == 0)
    def _even():
        _gather_into(tile_a_ref)
        _ln_from(tile_b_ref)

    @pl.when(slot == 1)
    def _odd():
        _gather_into(tile_b_ref)
        _ln_from(tile_a_ref)


def _embeddings_fused(input_ids, emb_table, rel_emb, gamma, beta, *, eps, tq=1024):
    B, S = input_ids.shape
    V, H = emb_table.shape
    R, _ = rel_emb.shape
    N = B * S
    assert H % 128 == 0, "hidden size must be lane-tile aligned"
    p = H // 128

    tq_eff = min(tq, _round_up(N, 8))
    n_pad = _round_up(N, tq_eff)
    stride = tq_eff + 8

    ids = jnp.clip(input_ids.reshape(N).astype(jnp.int32), 0, V - 1) * p
    ids = jnp.pad(ids, (0, n_pad - N + tq_eff))

    table_bytes = V * H * 4
    vmem_limit = min(
        2 * table_bytes + 12 * tq_eff * H * 4 + (12 << 20),
        60 << 20,
    )

    n_tiles = n_pad // tq_eff
    grid_spec = pltpu.PrefetchScalarGridSpec(
        num_scalar_prefetch=1,
        grid=(n_tiles + 1,),
        in_specs=[
            pl.BlockSpec(memory_space=pl.ANY),
            pl.BlockSpec((R, H), lambda i, ids_ref: (0, 0)),
            pl.BlockSpec((1, H), lambda i, ids_ref: (0, 0)),
            pl.BlockSpec((1, H), lambda i, ids_ref: (0, 0)),
        ],
        out_specs=[
            pl.BlockSpec(
                (tq_eff, H), lambda i, ids_ref: (jnp.maximum(i - 1, 0), 0)
            ),
            pl.BlockSpec((R, H), lambda i, ids_ref: (0, 0)),
        ],
        scratch_shapes=[
            pltpu.VMEM((V, H), jnp.float32),
            pltpu.VMEM((V * p, 128), jnp.float32),
            pltpu.VMEM((stride * p, 128), jnp.float32),
            pltpu.VMEM((stride * p, 128), jnp.float32),
            pltpu.SemaphoreType.DMA((8,)),
        ],
    )

    word, rel = pl.pallas_call(
        functools.partial(
            _fused_kernel,
            tq=tq_eff, p=p, v=V, stride=stride, n_tiles=n_tiles, eps=eps,
        ),
        out_shape=[
            jax.ShapeDtypeStruct((n_pad, H), jnp.float32),
            jax.ShapeDtypeStruct((R, H), jnp.float32),
        ],
        grid_spec=grid_spec,
        compiler_params=pltpu.CompilerParams(
            dimension_semantics=("arbitrary",),
            vmem_limit_bytes=vmem_limit,
        ),
    )(ids, emb_table, rel_emb, gamma.reshape(1, H), beta.reshape(1, H))
    return word[:N].reshape(B, S, H), rel


def kernel(input_ids, word_emb, rel_emb, rel_gamma, rel_beta):
    return _embeddings_fused(
        input_ids, word_emb, rel_emb, rel_gamma, rel_beta, eps=1e-7
    )
